# baseline (speedup 1.0000x reference)
import functools

import jax
import jax.numpy as jnp
from jax import lax
from jax.experimental import pallas as pl
from jax.experimental.pallas import tpu as pltpu
from jax.experimental.pallas import tpu_sc as plsc

B = 4096
CTX_DIM = 512
NUM_CLASS = 100000
N_CLS_CTX = 4
CLIP_CTX_LEN = 77
PRE = 5
SUF = CLIP_CTX_LEN - PRE - N_CLS_CTX
PRE_F = PRE * CTX_DIM
CLS_F = N_CLS_CTX * CTX_DIM
SUF_F = SUF * CTX_DIM
ROW_F = CLIP_CTX_LEN * CTX_DIM

NC = 2
NS = 16
NW = NC * NS
BPW = B // NW
CHUNK = 16
NCHUNK = BPW // CHUNK


def _sc_body(label_hbm, cls_hbm, pref_hbm, suf_hbm, out_hbm,
             idx_v, rows_v, pref_rep, suf_rep,
             sem_stage, sem_g0, sem_g1, sem_c0, sem_c1, sem_p, sem_s):
    cid = lax.axis_index("c")
    sid = lax.axis_index("s")
    wid = sid * NC + cid
    base = wid * BPW

    pltpu.async_copy(pref_hbm.at[0], pref_rep.at[sid], sem_stage).wait()
    pltpu.async_copy(suf_hbm.at[0], suf_rep.at[sid], sem_stage).wait()

    for c in range(NCHUNK):
        pltpu.async_copy(label_hbm.at[pl.ds(base + c * CHUNK, CHUNK)],
                         idx_v.at[c], sem_stage).wait()

    plsc.subcore_barrier()

    sem_g = (sem_g0, sem_g1)
    sem_c = (sem_c0, sem_c1)

    gather_h = [None, None]
    write_h = [None, None]
    pref_h = []
    suf_h = []
    gather_h[0] = pltpu.async_copy(cls_hbm.at[idx_v.at[0]], rows_v.at[0],
                                   sem_g[0])
    for c in range(NCHUNK):
        p = c % 2
        gather_h[p].wait()
        if c + 1 < NCHUNK:
            q = (c + 1) % 2
            if write_h[q] is not None:
                write_h[q].wait()
            gather_h[q] = pltpu.async_copy(cls_hbm.at[idx_v.at[c + 1]],
                                           rows_v.at[q], sem_g[q])
        b0 = base + c * CHUNK
        pref_h.append(pltpu.async_copy(
            pref_rep, out_hbm.at[pl.ds(b0, CHUNK), pl.ds(0, PRE_F)], sem_p))
        write_h[p] = pltpu.async_copy(
            rows_v.at[p], out_hbm.at[pl.ds(b0, CHUNK), pl.ds(PRE_F, CLS_F)],
            sem_c[p])
        suf_h.append(pltpu.async_copy(
            suf_rep, out_hbm.at[pl.ds(b0, CHUNK), pl.ds(PRE_F + CLS_F, SUF_F)],
            sem_s))

    for h in pref_h:
        h.wait()
    for h in suf_h:
        h.wait()
    write_h[0].wait()
    write_h[1].wait()
    plsc.subcore_barrier()


@functools.partial(
    pl.kernel,
    out_type=jax.ShapeDtypeStruct((B, ROW_F), jnp.float32),
    mesh=plsc.VectorSubcoreMesh(core_axis_name="c", subcore_axis_name="s"),
    scratch_types=[
        pltpu.VMEM((NCHUNK, CHUNK), jnp.int32),
        pltpu.VMEM((2, CHUNK, CLS_F), jnp.float32),
        pltpu.VMEM_SHARED((CHUNK, PRE_F), jnp.float32),
        pltpu.VMEM_SHARED((CHUNK, SUF_F), jnp.float32),
        pltpu.SemaphoreType.DMA,
        pltpu.SemaphoreType.DMA,
        pltpu.SemaphoreType.DMA,
        pltpu.SemaphoreType.DMA,
        pltpu.SemaphoreType.DMA,
        pltpu.SemaphoreType.DMA,
        pltpu.SemaphoreType.DMA,
    ],
)
def _prompt_concat_sc(label_hbm, cls_hbm, pref_hbm, suf_hbm, out_hbm, *scratch):
    _sc_body(label_hbm, cls_hbm, pref_hbm, suf_hbm, out_hbm, *scratch)


def kernel(label, view_label, time_label, cls_ctx, token_prefix, token_suffix):
    del view_label, time_label
    out2 = _prompt_concat_sc(
        label.astype(jnp.int32),
        cls_ctx.reshape(NUM_CLASS, CLS_F),
        token_prefix.reshape(1, PRE_F),
        token_suffix.reshape(1, SUF_F),
    )
    return out2.reshape(B, CLIP_CTX_LEN, CTX_DIM)


# speedup vs baseline: 1.0044x; 1.0044x over previous
import functools

import jax
import jax.numpy as jnp
from jax import lax
from jax.experimental import pallas as pl
from jax.experimental.pallas import tpu as pltpu
from jax.experimental.pallas import tpu_sc as plsc

B = 4096
CTX_DIM = 512
NUM_CLASS = 100000
N_CLS_CTX = 4
CLIP_CTX_LEN = 77
PRE = 5
SUF = CLIP_CTX_LEN - PRE - N_CLS_CTX

NC = 2
NS = 16
NW = NC * NS
BPW = B // NW
CHUNK = 16
NCHUNK = BPW // CHUNK


def _sc_body(label_hbm, cls_hbm, pref_hbm, suf_hbm, out_hbm,
             idx_v, rows_v, pref_rep, suf_rep,
             sem_stage, sem_g0, sem_g1, sem_c0, sem_c1, sem_p, sem_s):
    cid = lax.axis_index("c")
    sid = lax.axis_index("s")
    wid = sid * NC + cid
    base = wid * BPW

    pltpu.async_copy(pref_hbm.at[0], pref_rep.at[sid], sem_stage).wait()
    pltpu.async_copy(suf_hbm.at[0], suf_rep.at[sid], sem_stage).wait()

    for c in range(NCHUNK):
        pltpu.async_copy(label_hbm.at[pl.ds(base + c * CHUNK, CHUNK)],
                         idx_v.at[c], sem_stage).wait()

    plsc.subcore_barrier()

    sem_g = (sem_g0, sem_g1)
    sem_c = (sem_c0, sem_c1)

    gather_h = [None, None]
    write_h = [None, None]
    pref_h = []
    suf_h = []
    gather_h[0] = pltpu.async_copy(cls_hbm.at[idx_v.at[0]], rows_v.at[0],
                                   sem_g[0])
    for c in range(NCHUNK):
        p = c % 2
        gather_h[p].wait()
        if c + 1 < NCHUNK:
            q = (c + 1) % 2
            if write_h[q] is not None:
                write_h[q].wait()
            gather_h[q] = pltpu.async_copy(cls_hbm.at[idx_v.at[c + 1]],
                                           rows_v.at[q], sem_g[q])
        b0 = base + c * CHUNK
        pref_h.append(pltpu.async_copy(
            pref_rep, out_hbm.at[pl.ds(b0, CHUNK), pl.ds(0, PRE)], sem_p))
        write_h[p] = pltpu.async_copy(
            rows_v.at[p], out_hbm.at[pl.ds(b0, CHUNK), pl.ds(PRE, N_CLS_CTX)],
            sem_c[p])
        suf_h.append(pltpu.async_copy(
            suf_rep, out_hbm.at[pl.ds(b0, CHUNK), pl.ds(PRE + N_CLS_CTX, SUF)],
            sem_s))

    for h in pref_h:
        h.wait()
    for h in suf_h:
        h.wait()
    write_h[0].wait()
    write_h[1].wait()
    plsc.subcore_barrier()


@functools.partial(
    pl.kernel,
    out_type=jax.ShapeDtypeStruct((B, CLIP_CTX_LEN, CTX_DIM), jnp.float32),
    mesh=plsc.VectorSubcoreMesh(core_axis_name="c", subcore_axis_name="s"),
    compiler_params=pltpu.CompilerParams(use_tc_tiling_on_sc=False),
    scratch_types=[
        pltpu.VMEM((NCHUNK, CHUNK), jnp.int32),
        pltpu.VMEM((2, CHUNK, N_CLS_CTX, CTX_DIM), jnp.float32),
        pltpu.VMEM_SHARED((CHUNK, PRE, CTX_DIM), jnp.float32),
        pltpu.VMEM_SHARED((CHUNK, SUF, CTX_DIM), jnp.float32),
        pltpu.SemaphoreType.DMA,
        pltpu.SemaphoreType.DMA,
        pltpu.SemaphoreType.DMA,
        pltpu.SemaphoreType.DMA,
        pltpu.SemaphoreType.DMA,
        pltpu.SemaphoreType.DMA,
        pltpu.SemaphoreType.DMA,
    ],
)
def _prompt_concat_sc(label_hbm, cls_hbm, pref_hbm, suf_hbm, out_hbm, *scratch):
    _sc_body(label_hbm, cls_hbm, pref_hbm, suf_hbm, out_hbm, *scratch)


def kernel(label, view_label, time_label, cls_ctx, token_prefix, token_suffix):
    del view_label, time_label
    return _prompt_concat_sc(label.astype(jnp.int32), cls_ctx,
                             token_prefix, token_suffix)


# speedup vs baseline: 5.9221x; 5.8962x over previous
import functools

import jax
import jax.numpy as jnp
from jax import lax
from jax.experimental import pallas as pl
from jax.experimental.pallas import tpu as pltpu
from jax.experimental.pallas import tpu_sc as plsc

B = 4096
D = 512
NUM_CLASS = 100000
N_CLS = 4
PRE = 5
T = 77
SUF = T - PRE - N_CLS
NSTAT = PRE + SUF

NC = 2
NS = 16
NW = NC * NS
BPW = B // NW
CHUNK = 8
NCHUNK = BPW // CHUNK
REP = 32
NSEG = B // REP
LANES = 16
MPD = D // LANES


def _fill_rep(static_v, rep_v, task):
    row = [static_v[task, pl.ds(m * LANES, LANES)] for m in range(MPD)]

    def body(j, _):
        for m in range(MPD):
            rep_v[j, pl.ds(m * LANES, LANES)] = row[m]
        return 0

    lax.fori_loop(0, REP, body, 0)


def _bcast_plane(static_v, rep_v, out_hbm, task, sem_b):
    token = jnp.where(task >= PRE, task + N_CLS, task)
    _fill_rep(static_v, rep_v, task)

    def enq(seg, _):
        b0 = pl.multiple_of(seg * REP, 8)
        pltpu.async_copy(rep_v, out_hbm.at[token, pl.ds(b0, REP)], sem_b)
        return 0

    lax.fori_loop(0, NSEG, enq, 0)


def _drain_bcast(rep_v, out_hbm, sem_b):
    def drn(seg, _):
        pltpu.make_async_copy(rep_v, out_hbm.at[0, pl.ds(0, REP)], sem_b).wait()
        return 0

    lax.fori_loop(0, NSEG, drn, 0)


def _sc_body(label_hbm, static_hbm, cls_hbm, out_hbm,
             static_v, rep_v, rows_v, asm_v, idx_v,
             sem_st, sem_lb, sem_g0, sem_g1, sem_a, sem_b):
    cid = lax.axis_index("c")
    sid = lax.axis_index("s")
    wid = sid * NC + cid
    base = wid * BPW
    sem_g = (sem_g0, sem_g1)

    pltpu.async_copy(static_hbm, static_v, sem_st)
    pltpu.async_copy(label_hbm.at[pl.ds(base, BPW)], idx_v, sem_lb).wait()
    pltpu.make_async_copy(static_hbm, static_v, sem_st).wait()

    pltpu.async_copy(cls_hbm.at[idx_v.at[pl.ds(0, CHUNK)]], rows_v.at[0],
                     sem_g[0])
    pltpu.async_copy(cls_hbm.at[idx_v.at[pl.ds(CHUNK, CHUNK)]], rows_v.at[1],
                     sem_g[1])

    _bcast_plane(static_v, rep_v, out_hbm, wid, sem_b)

    def pair_body(k, _):
        for p in range(2):
            c = 2 * k + p
            pltpu.make_async_copy(cls_hbm.at[idx_v.at[pl.ds(0, CHUNK)]],
                                  rows_v.at[p], sem_g[p]).wait()

            def asm_body(bi, _, p=p):
                for r in range(N_CLS):
                    for ct in range(4):
                        for m in range(8):
                            v = rows_v[p, bi, ct * 4 + r,
                                       pl.ds(m * LANES, LANES)]
                            asm_v[r, bi,
                                  pl.ds(ct * 128 + m * LANES, LANES)] = v
                return 0

            lax.fori_loop(0, CHUNK, asm_body, 0)

            @pl.when(c + 2 < NCHUNK)
            def _(p=p, c=c):
                i0 = pl.multiple_of((c + 2) * CHUNK, 8)
                pltpu.async_copy(cls_hbm.at[idx_v.at[pl.ds(i0, CHUNK)]],
                                 rows_v.at[p], sem_g[p])

            b0 = pl.multiple_of(base + c * CHUNK, 8)
            for r in range(N_CLS):
                pltpu.async_copy(asm_v.at[r],
                                 out_hbm.at[PRE + r, pl.ds(b0, CHUNK)], sem_a)
            for r in range(N_CLS):
                pltpu.make_async_copy(asm_v.at[0],
                                      out_hbm.at[PRE, pl.ds(base, CHUNK)],
                                      sem_a).wait()
        return 0

    lax.fori_loop(0, NCHUNK // 2, pair_body, 0)

    _drain_bcast(rep_v, out_hbm, sem_b)
    _bcast_plane(static_v, rep_v, out_hbm, wid + NW, sem_b)

    @pl.when(wid + 2 * NW < NSTAT)
    def _():
        _drain_bcast(rep_v, out_hbm, sem_b)
        _bcast_plane(static_v, rep_v, out_hbm, wid + 2 * NW, sem_b)

    _drain_bcast(rep_v, out_hbm, sem_b)


@functools.partial(
    pl.kernel,
    out_type=jax.ShapeDtypeStruct((T, B, D), jnp.float32),
    mesh=plsc.VectorSubcoreMesh(core_axis_name="c", subcore_axis_name="s"),
    scratch_types=[
        pltpu.VMEM((NSTAT, D), jnp.float32),
        pltpu.VMEM((REP, D), jnp.float32),
        pltpu.VMEM((2, CHUNK, 16, 128), jnp.float32),
        pltpu.VMEM((N_CLS, CHUNK, D), jnp.float32),
        pltpu.VMEM((BPW,), jnp.int32),
        pltpu.SemaphoreType.DMA,
        pltpu.SemaphoreType.DMA,
        pltpu.SemaphoreType.DMA,
        pltpu.SemaphoreType.DMA,
        pltpu.SemaphoreType.DMA,
        pltpu.SemaphoreType.DMA,
    ],
)
def _prompt_concat_sc(label_hbm, static_hbm, cls_hbm, out_hbm, *scratch):
    _sc_body(label_hbm, static_hbm, cls_hbm, out_hbm, *scratch)


def kernel(label, view_label, time_label, cls_ctx, token_prefix, token_suffix):
    del view_label, time_label
    static_rows = jnp.concatenate(
        [token_prefix.reshape(PRE, D), token_suffix.reshape(SUF, D)], axis=0)
    cls_p = (cls_ctx.reshape(NUM_CLASS, N_CLS, 4, 128)
             .transpose(0, 2, 1, 3).reshape(NUM_CLASS, 16, 128))
    out_t = _prompt_concat_sc(label.astype(jnp.int32), static_rows, cls_p)
    return out_t.transpose(1, 0, 2)
